# baseline (speedup 1.0000x reference)
import functools
import jax
import jax.numpy as jnp
from jax import lax
from jax.experimental import pallas as pl
from jax.experimental.pallas import tpu as pltpu
from jax.experimental.pallas import tpu_sc as plsc

N = 10000
NP = 10240
E = 320000
G = 64
HID = 64
NHEADS = 4
NSTEPS = 3
ALPHA = 0.2
RW = 128
RSROWS = 160
CH = 128
TROWS = 160
NGROUPS = 20
EROWS = 16 * TROWS
EPAD = EROWS * CH
BN = 1024

_f32 = jnp.float32
_i32 = jnp.int32


def _emb_body(x_ref, w_ref, b_ref, o_ref):
    i = pl.program_id(0)
    v = jnp.dot(x_ref[...], w_ref[...], preferred_element_type=_f32) + b_ref[...]
    r = lax.broadcasted_iota(_i32, (BN, 1), 0) + i * BN
    o_ref[...] = jnp.where(r < N, v, 0.0)


def _embed(xp, W_emb, b_emb):
    return pl.pallas_call(
        _emb_body,
        grid=(NP // BN,),
        in_specs=[
            pl.BlockSpec((BN, 128), lambda i: (i, 0)),
            pl.BlockSpec((128, HID), lambda i: (0, 0)),
            pl.BlockSpec((1, HID), lambda i: (0, 0)),
        ],
        out_specs=pl.BlockSpec((BN, HID), lambda i: (i, 0)),
        out_shape=jax.ShapeDtypeStruct((NP, HID), _f32),
    )(xp, W_emb, b_emb.reshape(1, HID))


def _pa_body(x_ref, wbig_ref, va_ref, hc_ref, ss_ref):
    xb = x_ref[...]
    hc_ref[...] = jnp.dot(xb, wbig_ref[0], preferred_element_type=_f32)[None]
    ss_ref[...] = jnp.dot(xb, va_ref[0], preferred_element_type=_f32)[None]


def _phase_a(xc, Wbig, Vbig):
    return pl.pallas_call(
        _pa_body,
        grid=(2, NP // BN),
        in_specs=[
            pl.BlockSpec((BN, HID), lambda c, i: (i, 0)),
            pl.BlockSpec((1, HID, RW), lambda c, i: (c, 0, 0)),
            pl.BlockSpec((1, HID, 4), lambda c, i: (c, 0, 0)),
        ],
        out_specs=[
            pl.BlockSpec((1, BN, RW), lambda c, i: (c, i, 0)),
            pl.BlockSpec((1, BN, 4), lambda c, i: (c, i, 0)),
        ],
        out_shape=[
            jax.ShapeDtypeStruct((2, NP, RW), _f32),
            jax.ShapeDtypeStruct((2, NP, 4), _f32),
        ],
    )(xc, Wbig, Vbig)


def _update(a, d, xb, yb, i):
    s = jnp.zeros_like(xb)
    for k in range(NHEADS):
        c, p = k // 2, k % 2
        num = a[c, :, p * HID:(p + 1) * HID]
        den = d[c, :, p:p + 1]
        hp = num / (den + 1e-16) + xb
        s = s + jnp.where(hp > 0, hp, jnp.exp(hp) - 1.0)
    att = s * (1.0 / NHEADS)
    xn = xb + 1.0 * (att - 1.0 * xb - 1.0 * yb)
    yn = yb + 1.0 * xn
    r = lax.broadcasted_iota(_i32, (BN, 1), 0) + i * BN
    m = r < N
    return jnp.where(m, xn, 0.0), jnp.where(m, yn, 0.0)


def _pc_body(acc_ref, den_ref, x_ref, y_ref, xo_ref, yo_ref):
    i = pl.program_id(0)
    xn, yn = _update(acc_ref[...], den_ref[...], x_ref[...], y_ref[...], i)
    xo_ref[...] = xn
    yo_ref[...] = yn


def _phase_c(acc, den, xc, yc):
    return pl.pallas_call(
        _pc_body,
        grid=(NP // BN,),
        in_specs=[
            pl.BlockSpec((2, BN, RW), lambda i: (0, i, 0)),
            pl.BlockSpec((2, BN, 2), lambda i: (0, i, 0)),
            pl.BlockSpec((BN, HID), lambda i: (i, 0)),
            pl.BlockSpec((BN, HID), lambda i: (i, 0)),
        ],
        out_specs=[
            pl.BlockSpec((BN, HID), lambda i: (i, 0)),
            pl.BlockSpec((BN, HID), lambda i: (i, 0)),
        ],
        out_shape=[
            jax.ShapeDtypeStruct((NP, HID), _f32),
            jax.ShapeDtypeStruct((NP, HID), _f32),
        ],
    )(acc, den, xc, yc)


def _pcd_body(acc_ref, den_ref, x_ref, y_ref, wo_ref, bo_ref, b_ref, po_ref):
    i = pl.program_id(0)
    _, yn = _update(acc_ref[...], den_ref[...], x_ref[...], y_ref[...], i)
    z = jnp.sum(yn * wo_ref[...], axis=1, keepdims=True) + bo_ref[...]
    r = lax.broadcasted_iota(_i32, (BN, 1), 0) + i * BN
    z = jnp.where(r < N, z, 0.0)
    gi = lax.broadcasted_iota(_i32, (BN, G), 1)
    contrib = jnp.where(b_ref[...] == gi, z, 0.0)
    part = jnp.sum(contrib, axis=0, keepdims=True)

    @pl.when(i == 0)
    def _init():
        po_ref[...] = jnp.zeros((1, G), _f32)
    po_ref[...] += part


def _phase_cd(acc, den, xc, yc, W_out, b_out, batchp):
    return pl.pallas_call(
        _pcd_body,
        grid=(NP // BN,),
        in_specs=[
            pl.BlockSpec((2, BN, RW), lambda i: (0, i, 0)),
            pl.BlockSpec((2, BN, 2), lambda i: (0, i, 0)),
            pl.BlockSpec((BN, HID), lambda i: (i, 0)),
            pl.BlockSpec((BN, HID), lambda i: (i, 0)),
            pl.BlockSpec((1, HID), lambda i: (0, 0)),
            pl.BlockSpec((1, 1), lambda i: (0, 0)),
            pl.BlockSpec((BN, 1), lambda i: (i, 0)),
        ],
        out_specs=pl.BlockSpec((1, G), lambda i: (0, 0)),
        out_shape=jax.ShapeDtypeStruct((1, G), _f32),
    )(acc, den, xc, yc, W_out.reshape(1, HID), b_out.reshape(1, 1), batchp)


def _sc_w_body(ss_ref, src_ref, dst_ref, w_out_ref, rs_out_ref,
               ssv, sbuf, dbuf, wo, rs2d, idb, idb2, accrs):
    c = lax.axis_index("c")
    s = lax.axis_index("s")

    pltpu.sync_copy(ss_ref.at[c], ssv)

    def _zrs(r, _):
        for q in range(128 // 16):
            rs2d[r, pl.ds(q * 16, 16)] = jnp.zeros((16,), _f32)
        return 0
    lax.fori_loop(0, RSROWS, _zrs, 0)

    @pl.when(s < 10)
    def _zacc():
        pltpu.sync_copy(rs2d.at[pl.ds(s * 16, 16)], accrs.at[pl.ds(s * 16, 16)])

    lane = lax.iota(_i32, 16)
    for t in range(8):
        idb[0, pl.ds(t * 16, 16)] = lane + t * 16
    for t in range(2):
        idb2[0, pl.ds(t * 16, 16)] = lane + (CH + t * 16)
    plsc.subcore_barrier()

    def _group(g, _):
        row0 = s * TROWS + g * 8
        pltpu.sync_copy(src_ref.at[pl.ds(row0, 8)], sbuf)
        pltpu.sync_copy(dst_ref.at[pl.ds(row0, 8)], dbuf)
        for j in range(8):
            for t in range(8):
                sv = sbuf[j, pl.ds(t * 16, 16)]
                sv4 = sv * 4
                dv4 = dbuf[j, pl.ds(t * 16, 16)] * 4
                as_a = plsc.load_gather(ssv, [sv4])
                as_b = plsc.load_gather(ssv, [sv4 + 1])
                ad_a = plsc.load_gather(ssv, [dv4 + 2])
                ad_b = plsc.load_gather(ssv, [dv4 + 3])
                e_a = as_a + ad_a
                e_b = as_b + ad_b
                w_a = jnp.exp(-jnp.maximum(e_a, ALPHA * e_a))
                w_b = jnp.exp(-jnp.maximum(e_b, ALPHA * e_b))
                widx = (lane + t * 16) * 2 + j * 2 * CH
                plsc.store_scatter(wo, [widx], w_a)
                plsc.store_scatter(wo, [widx + 1], w_b)
                rowv = lax.shift_right_logical(sv, 6)
                colv = (sv * 2) & 127
                plsc.addupdate_scatter(rs2d, [rowv, colv], w_a)
                plsc.addupdate_scatter(rs2d, [rowv, colv + 1], w_b)
        pltpu.sync_copy(wo, w_out_ref.at[c, pl.ds(row0 * 2 * CH, 8 * 2 * CH)])
        return 0

    lax.fori_loop(0, NGROUPS, _group, 0)

    pltpu.sync_copy(rs2d.at[pl.ds(0, CH)], accrs.at[idb.at[0]], add=True)
    pltpu.sync_copy(rs2d.at[pl.ds(CH, RSROWS - CH)], accrs.at[idb2.at[0]],
                    add=True)
    plsc.subcore_barrier()

    @pl.when(s < 10)
    def _copyout():
        pltpu.sync_copy(accrs.at[pl.ds(s * 16, 16)], rs2d.at[pl.ds(0, 16)])
        pltpu.sync_copy(rs2d.at[pl.ds(0, 16)],
                        rs_out_ref.at[c, pl.ds(s * 16, 16)])


def _sc_agg_body(hc_ref, w_ref, src_ref, dst_ref, out_ref,
                 sbuf, dbuf, wg, ibuf0, ibuf1, hbuf0, hbuf1, acc,
                 sem0, sem1, ssem0, ssem1):
    c = lax.axis_index("c")
    s = lax.axis_index("s")
    cN = c * NP

    def _z(r, _):
        for q in range(RW // 16):
            hbuf0[r, pl.ds(q * 16, 16)] = jnp.zeros((16,), _f32)
        return 0
    lax.fori_loop(0, CH, _z, 0)
    for r in range(5):
        pltpu.sync_copy(hbuf0, acc.at[pl.ds(s * 640 + r * CH, CH)])
    plsc.subcore_barrier()

    def _build_idx(j, ib):
        for t in range(8):
            dv = dbuf[j, pl.ds(t * 16, 16)]
            ib[pl.ds(t * 16, 16)] = dv + cN

    def _group(g, _):
        row0 = s * TROWS + g * 8
        pltpu.sync_copy(src_ref.at[pl.ds(row0, 8)], sbuf)
        pltpu.sync_copy(dst_ref.at[pl.ds(row0, 8)], dbuf)
        pltpu.sync_copy(w_ref.at[c, pl.ds(row0 * 2 * CH, 8 * 2 * CH)], wg)
        bufs = [(ibuf0, hbuf0, sem0, ssem0), (ibuf1, hbuf1, sem1, ssem1)]
        scat = [None, None]

        def _fire_gather(ib, hb, sem):
            return (
                pltpu.async_copy(hc_ref.at[ib.at[pl.ds(0, 64)]],
                                 hb.at[pl.ds(0, 64)], sem),
                pltpu.async_copy(hc_ref.at[ib.at[pl.ds(64, 64)]],
                                 hb.at[pl.ds(64, 64)], sem),
            )

        _build_idx(0, ibuf0)
        pending = _fire_gather(ibuf0, hbuf0, sem0)
        for j in range(8):
            _, hb, _, ssem = bufs[j % 2]
            nxt = None
            if j < 7:
                b2 = (j + 1) % 2
                ib2, hb2, sem2, _ = bufs[b2]
                if scat[b2] is not None:
                    scat[b2].wait()
                _build_idx(j + 1, ib2)
                nxt = _fire_gather(ib2, hb2, sem2)
            pending[0].wait()
            pending[1].wait()

            @plsc.parallel_loop(0, CH, unroll=4)
            def _mrow(e):
                wae = plsc.load_gather(wg, [jnp.full((16,), j * 2 * CH + 2 * e,
                                                     _i32)])
                wbe = plsc.load_gather(wg, [jnp.full((16,), j * 2 * CH + 2 * e
                                                     + 1, _i32)])
                for q in range(4):
                    hb[e, pl.ds(q * 16, 16)] = hb[e, pl.ds(q * 16, 16)] * wae
                for q in range(4, 8):
                    hb[e, pl.ds(q * 16, 16)] = hb[e, pl.ds(q * 16, 16)] * wbe

            scat[j % 2] = pltpu.async_copy(hb, acc.at[sbuf.at[j]], ssem,
                                           add=True)
            pending = nxt
        scat[0].wait()
        scat[1].wait()
        return 0

    lax.fori_loop(0, NGROUPS, _group, 0)
    plsc.subcore_barrier()

    for r in range(5):
        b = s * 640 + r * CH
        pltpu.sync_copy(acc.at[pl.ds(b, CH)], hbuf0)
        pltpu.sync_copy(hbuf0, out_ref.at[c, pl.ds(b, CH)])


_SC_PARAMS = pltpu.CompilerParams(needs_layout_passes=False)
_SC_MESH = dict(core_axis_name="c", subcore_axis_name="s",
                num_cores=2, num_subcores=16)


@functools.cache
def _make_sc_w():
  return pl.kernel(
    _sc_w_body,
    out_type=[
        jax.ShapeDtypeStruct((2, EPAD * 2), _f32),
        jax.ShapeDtypeStruct((2, RSROWS, 128), _f32),
    ],
    mesh=plsc.VectorSubcoreMesh(**_SC_MESH),
    compiler_params=_SC_PARAMS,
    scratch_types=[
        pltpu.VMEM((NP * 4,), _f32),
        pltpu.VMEM((8, CH), _i32),
        pltpu.VMEM((8, CH), _i32),
        pltpu.VMEM((8 * 2 * CH,), _f32),
        pltpu.VMEM((RSROWS, 128), _f32),
        pltpu.VMEM((1, CH), _i32),
        pltpu.VMEM((1, 32), _i32),
        pltpu.VMEM_SHARED((RSROWS, 128), _f32),
    ],
  )


@functools.cache
def _make_sc_agg():
  return pl.kernel(
    _sc_agg_body,
    out_type=jax.ShapeDtypeStruct((2, NP, RW), _f32),
    mesh=plsc.VectorSubcoreMesh(**_SC_MESH),
    compiler_params=_SC_PARAMS,
    scratch_types=[
        pltpu.VMEM((8, CH), _i32),
        pltpu.VMEM((8, CH), _i32),
        pltpu.VMEM((8 * 2 * CH,), _f32),
        pltpu.VMEM((CH,), _i32),
        pltpu.VMEM((CH,), _i32),
        pltpu.VMEM((CH, RW), _f32),
        pltpu.VMEM((CH, RW), _f32),
        pltpu.VMEM_SHARED((NP, RW), _f32),
        pltpu.SemaphoreType.DMA,
        pltpu.SemaphoreType.DMA,
        pltpu.SemaphoreType.DMA,
        pltpu.SemaphoreType.DMA,
    ],
  )


def kernel(x, edge_index, batch, W_emb, b_emb, W_att, a_att, W_out, b_out):
    xp = jnp.pad(x, ((0, NP - N), (0, 0)))
    src = jnp.concatenate([edge_index[0],
                           jnp.full((EPAD - E,), N, _i32)]).reshape(EROWS, CH)
    dst = jnp.concatenate([edge_index[1],
                           jnp.zeros((EPAD - E,), _i32)]).reshape(EROWS, CH)
    batchp = jnp.pad(batch, (0, NP - N)).reshape(NP, 1)

    va = jnp.einsum('kij,kj->ki', W_att, a_att[:, :HID])
    vd = jnp.einsum('kij,kj->ki', W_att, a_att[:, HID:])
    Wbig = jnp.stack([jnp.concatenate(
        [W_att[2 * c], W_att[2 * c + 1]], axis=1)
        for c in range(2)])
    Vbig = jnp.stack([jnp.stack(
        [va[2 * c], va[2 * c + 1], vd[2 * c], vd[2 * c + 1]], axis=1)
        for c in range(2)])

    xc = _embed(xp, W_emb, b_emb)
    yc = xc
    for step in range(NSTEPS):
        hc, ss = _phase_a(xc, Wbig, Vbig)
        wv, rs = _make_sc_w()(ss.reshape(2, NP * 4), src, dst)
        acc = _make_sc_agg()(hc.reshape(2 * NP, RW), wv, src, dst)
        den = rs.reshape(2, NP, 2)
        if step < NSTEPS - 1:
            xc, yc = _phase_c(acc, den, xc, yc)
        else:
            pooled = _phase_cd(acc, den, xc, yc, W_out, b_out, batchp)
    return pooled[0]


# speedup vs baseline: 1.0301x; 1.0301x over previous
import functools
import jax
import jax.numpy as jnp
from jax import lax
from jax.experimental import pallas as pl
from jax.experimental.pallas import tpu as pltpu
from jax.experimental.pallas import tpu_sc as plsc

N = 10000
NP = 10240
E = 320000
G = 64
HID = 64
NHEADS = 4
NSTEPS = 3
ALPHA = 0.2
RW = 128
RSROWS = 160
CH = 128
TROWS = 160
NGROUPS = 20
EROWS = 16 * TROWS
EPAD = EROWS * CH
BN = 1024

_f32 = jnp.float32
_i32 = jnp.int32


def _emb_body(x_ref, w_ref, b_ref, o_ref):
    i = pl.program_id(0)
    v = jnp.dot(x_ref[...], w_ref[...], preferred_element_type=_f32) + b_ref[...]
    r = lax.broadcasted_iota(_i32, (BN, 1), 0) + i * BN
    o_ref[...] = jnp.where(r < N, v, 0.0)


def _embed(xp, W_emb, b_emb):
    return pl.pallas_call(
        _emb_body,
        grid=(NP // BN,),
        in_specs=[
            pl.BlockSpec((BN, 128), lambda i: (i, 0)),
            pl.BlockSpec((128, HID), lambda i: (0, 0)),
            pl.BlockSpec((1, HID), lambda i: (0, 0)),
        ],
        out_specs=pl.BlockSpec((BN, HID), lambda i: (i, 0)),
        out_shape=jax.ShapeDtypeStruct((NP, HID), _f32),
    )(xp, W_emb, b_emb.reshape(1, HID))


def _pa_body(x_ref, wbig_ref, va_ref, hc_ref, ss_ref):
    xb = x_ref[...]
    hc_ref[...] = jnp.dot(xb, wbig_ref[0], preferred_element_type=_f32)[None]
    ss_ref[...] = jnp.dot(xb, va_ref[0], preferred_element_type=_f32)[None]


def _phase_a(xc, Wbig, Vbig):
    return pl.pallas_call(
        _pa_body,
        grid=(2, NP // BN),
        in_specs=[
            pl.BlockSpec((BN, HID), lambda c, i: (i, 0)),
            pl.BlockSpec((1, HID, RW), lambda c, i: (c, 0, 0)),
            pl.BlockSpec((1, HID, 4), lambda c, i: (c, 0, 0)),
        ],
        out_specs=[
            pl.BlockSpec((1, BN, RW), lambda c, i: (c, i, 0)),
            pl.BlockSpec((1, BN, 4), lambda c, i: (c, i, 0)),
        ],
        out_shape=[
            jax.ShapeDtypeStruct((2, NP, RW), _f32),
            jax.ShapeDtypeStruct((2, NP, 4), _f32),
        ],
    )(xc, Wbig, Vbig)


def _update(a, d, xb, yb, i):
    s = jnp.zeros_like(xb)
    for k in range(NHEADS):
        c, p = k // 2, k % 2
        num = a[c, :, p * HID:(p + 1) * HID]
        den = d[c, :, p:p + 1]
        hp = num / (den + 1e-16) + xb
        s = s + jnp.where(hp > 0, hp, jnp.exp(hp) - 1.0)
    att = s * (1.0 / NHEADS)
    xn = xb + 1.0 * (att - 1.0 * xb - 1.0 * yb)
    yn = yb + 1.0 * xn
    r = lax.broadcasted_iota(_i32, (BN, 1), 0) + i * BN
    m = r < N
    return jnp.where(m, xn, 0.0), jnp.where(m, yn, 0.0)


def _pc_body(acc_ref, den_ref, x_ref, y_ref, xo_ref, yo_ref):
    i = pl.program_id(0)
    xn, yn = _update(acc_ref[...], den_ref[...], x_ref[...], y_ref[...], i)
    xo_ref[...] = xn
    yo_ref[...] = yn


def _phase_c(acc, den, xc, yc):
    return pl.pallas_call(
        _pc_body,
        grid=(NP // BN,),
        in_specs=[
            pl.BlockSpec((2, BN, RW), lambda i: (0, i, 0)),
            pl.BlockSpec((2, BN, 2), lambda i: (0, i, 0)),
            pl.BlockSpec((BN, HID), lambda i: (i, 0)),
            pl.BlockSpec((BN, HID), lambda i: (i, 0)),
        ],
        out_specs=[
            pl.BlockSpec((BN, HID), lambda i: (i, 0)),
            pl.BlockSpec((BN, HID), lambda i: (i, 0)),
        ],
        out_shape=[
            jax.ShapeDtypeStruct((NP, HID), _f32),
            jax.ShapeDtypeStruct((NP, HID), _f32),
        ],
    )(acc, den, xc, yc)


def _pd_body(y_ref, wo_ref, bo_ref, b_ref, o_ref):
    yb = y_ref[...]
    z = jnp.sum(yb * wo_ref[...], axis=1, keepdims=True) + bo_ref[...]
    r = lax.broadcasted_iota(_i32, (NP, 1), 0)
    z = jnp.where(r < N, z, 0.0)
    gi = lax.broadcasted_iota(_i32, (NP, G), 1)
    contrib = jnp.where(b_ref[...] == gi, z, 0.0)
    o_ref[...] = jnp.sum(contrib, axis=0, keepdims=True)


def _phase_d(yc, W_out, b_out, batchp):
    return pl.pallas_call(
        _pd_body,
        grid=(1,),
        in_specs=[
            pl.BlockSpec((NP, HID), lambda i: (0, 0)),
            pl.BlockSpec((1, HID), lambda i: (0, 0)),
            pl.BlockSpec((1, 1), lambda i: (0, 0)),
            pl.BlockSpec((NP, 1), lambda i: (0, 0)),
        ],
        out_specs=pl.BlockSpec((1, G), lambda i: (0, 0)),
        out_shape=jax.ShapeDtypeStruct((1, G), _f32),
    )(yc, W_out.reshape(1, HID), b_out.reshape(1, 1), batchp)


def _sc_w_body(ss_ref, src_ref, dst_ref, w_out_ref, rs_out_ref,
               ssv, sbuf, dbuf, wo, rs2d, idb, idb2, accrs):
    c = lax.axis_index("c")
    s = lax.axis_index("s")

    pltpu.sync_copy(ss_ref.at[c], ssv)

    def _zrs(r, _):
        for q in range(128 // 16):
            rs2d[r, pl.ds(q * 16, 16)] = jnp.zeros((16,), _f32)
        return 0
    lax.fori_loop(0, RSROWS, _zrs, 0)

    @pl.when(s < 10)
    def _zacc():
        pltpu.sync_copy(rs2d.at[pl.ds(s * 16, 16)], accrs.at[pl.ds(s * 16, 16)])

    lane = lax.iota(_i32, 16)
    for t in range(8):
        idb[0, pl.ds(t * 16, 16)] = lane + t * 16
    for t in range(2):
        idb2[0, pl.ds(t * 16, 16)] = lane + (CH + t * 16)
    plsc.subcore_barrier()

    def _group(g, _):
        row0 = s * TROWS + g * 8
        pltpu.sync_copy(src_ref.at[pl.ds(row0, 8)], sbuf)
        pltpu.sync_copy(dst_ref.at[pl.ds(row0, 8)], dbuf)
        for j in range(8):
            for t in range(8):
                sv = sbuf[j, pl.ds(t * 16, 16)]
                sv4 = sv * 4
                dv4 = dbuf[j, pl.ds(t * 16, 16)] * 4
                as_a = plsc.load_gather(ssv, [sv4])
                as_b = plsc.load_gather(ssv, [sv4 + 1])
                ad_a = plsc.load_gather(ssv, [dv4 + 2])
                ad_b = plsc.load_gather(ssv, [dv4 + 3])
                e_a = as_a + ad_a
                e_b = as_b + ad_b
                w_a = jnp.exp(-jnp.maximum(e_a, ALPHA * e_a))
                w_b = jnp.exp(-jnp.maximum(e_b, ALPHA * e_b))
                widx = (lane + t * 16) * 2 + j * 2 * CH
                plsc.store_scatter(wo, [widx], w_a)
                plsc.store_scatter(wo, [widx + 1], w_b)
                rowv = lax.shift_right_logical(sv, 6)
                colv = (sv * 2) & 127
                plsc.addupdate_scatter(rs2d, [rowv, colv], w_a)
                plsc.addupdate_scatter(rs2d, [rowv, colv + 1], w_b)
        pltpu.sync_copy(wo, w_out_ref.at[c, pl.ds(row0 * 2 * CH, 8 * 2 * CH)])
        return 0

    lax.fori_loop(0, NGROUPS, _group, 0)

    pltpu.sync_copy(rs2d.at[pl.ds(0, CH)], accrs.at[idb.at[0]], add=True)
    pltpu.sync_copy(rs2d.at[pl.ds(CH, RSROWS - CH)], accrs.at[idb2.at[0]],
                    add=True)
    plsc.subcore_barrier()

    @pl.when(s < 10)
    def _copyout():
        pltpu.sync_copy(accrs.at[pl.ds(s * 16, 16)], rs2d.at[pl.ds(0, 16)])
        pltpu.sync_copy(rs2d.at[pl.ds(0, 16)],
                        rs_out_ref.at[c, pl.ds(s * 16, 16)])


def _sc_agg_body(hc_ref, w_ref, src_ref, dst_ref, out_ref,
                 sbuf, dbuf, wg, ibuf0, ibuf1, hbuf0, hbuf1, acc,
                 sem0, sem1, ssem0, ssem1):
    c = lax.axis_index("c")
    s = lax.axis_index("s")
    cN = c * NP

    def _z(r, _):
        for q in range(RW // 16):
            hbuf0[r, pl.ds(q * 16, 16)] = jnp.zeros((16,), _f32)
        return 0
    lax.fori_loop(0, CH, _z, 0)
    for r in range(5):
        pltpu.sync_copy(hbuf0, acc.at[pl.ds(s * 640 + r * CH, CH)])
    plsc.subcore_barrier()

    def _build_idx(j, ib):
        for t in range(8):
            dv = dbuf[j, pl.ds(t * 16, 16)]
            ib[pl.ds(t * 16, 16)] = dv + cN

    def _group(g, _):
        row0 = s * TROWS + g * 8
        pltpu.sync_copy(src_ref.at[pl.ds(row0, 8)], sbuf)
        pltpu.sync_copy(dst_ref.at[pl.ds(row0, 8)], dbuf)
        pltpu.sync_copy(w_ref.at[c, pl.ds(row0 * 2 * CH, 8 * 2 * CH)], wg)
        bufs = [(ibuf0, hbuf0, sem0, ssem0), (ibuf1, hbuf1, sem1, ssem1)]
        scat = [None, None]

        def _fire_gather(ib, hb, sem):
            return (
                pltpu.async_copy(hc_ref.at[ib.at[pl.ds(0, 64)]],
                                 hb.at[pl.ds(0, 64)], sem),
                pltpu.async_copy(hc_ref.at[ib.at[pl.ds(64, 64)]],
                                 hb.at[pl.ds(64, 64)], sem),
            )

        _build_idx(0, ibuf0)
        pending = _fire_gather(ibuf0, hbuf0, sem0)
        for j in range(8):
            _, hb, _, ssem = bufs[j % 2]
            nxt = None
            if j < 7:
                b2 = (j + 1) % 2
                ib2, hb2, sem2, _ = bufs[b2]
                if scat[b2] is not None:
                    scat[b2].wait()
                _build_idx(j + 1, ib2)
                nxt = _fire_gather(ib2, hb2, sem2)
            pending[0].wait()
            pending[1].wait()

            @plsc.parallel_loop(0, CH, unroll=4)
            def _mrow(e):
                wae = plsc.load_gather(wg, [jnp.full((16,), j * 2 * CH + 2 * e,
                                                     _i32)])
                wbe = plsc.load_gather(wg, [jnp.full((16,), j * 2 * CH + 2 * e
                                                     + 1, _i32)])
                for q in range(4):
                    hb[e, pl.ds(q * 16, 16)] = hb[e, pl.ds(q * 16, 16)] * wae
                for q in range(4, 8):
                    hb[e, pl.ds(q * 16, 16)] = hb[e, pl.ds(q * 16, 16)] * wbe

            scat[j % 2] = pltpu.async_copy(hb, acc.at[sbuf.at[j]], ssem,
                                           add=True)
            pending = nxt
        scat[0].wait()
        scat[1].wait()
        return 0

    lax.fori_loop(0, NGROUPS, _group, 0)
    plsc.subcore_barrier()

    for r in range(5):
        b = s * 640 + r * CH
        pltpu.sync_copy(acc.at[pl.ds(b, CH)], hbuf0)
        pltpu.sync_copy(hbuf0, out_ref.at[c, pl.ds(b, CH)])


_SC_PARAMS = pltpu.CompilerParams(needs_layout_passes=False)
_SC_MESH = dict(core_axis_name="c", subcore_axis_name="s",
                num_cores=2, num_subcores=16)


@functools.cache
def _make_sc_w():
  return pl.kernel(
    _sc_w_body,
    out_type=[
        jax.ShapeDtypeStruct((2, EPAD * 2), _f32),
        jax.ShapeDtypeStruct((2, RSROWS, 128), _f32),
    ],
    mesh=plsc.VectorSubcoreMesh(**_SC_MESH),
    compiler_params=_SC_PARAMS,
    scratch_types=[
        pltpu.VMEM((NP * 4,), _f32),
        pltpu.VMEM((8, CH), _i32),
        pltpu.VMEM((8, CH), _i32),
        pltpu.VMEM((8 * 2 * CH,), _f32),
        pltpu.VMEM((RSROWS, 128), _f32),
        pltpu.VMEM((1, CH), _i32),
        pltpu.VMEM((1, 32), _i32),
        pltpu.VMEM_SHARED((RSROWS, 128), _f32),
    ],
  )


@functools.cache
def _make_sc_agg():
  return pl.kernel(
    _sc_agg_body,
    out_type=jax.ShapeDtypeStruct((2, NP, RW), _f32),
    mesh=plsc.VectorSubcoreMesh(**_SC_MESH),
    compiler_params=_SC_PARAMS,
    scratch_types=[
        pltpu.VMEM((8, CH), _i32),
        pltpu.VMEM((8, CH), _i32),
        pltpu.VMEM((8 * 2 * CH,), _f32),
        pltpu.VMEM((CH,), _i32),
        pltpu.VMEM((CH,), _i32),
        pltpu.VMEM((CH, RW), _f32),
        pltpu.VMEM((CH, RW), _f32),
        pltpu.VMEM_SHARED((NP, RW), _f32),
        pltpu.SemaphoreType.DMA,
        pltpu.SemaphoreType.DMA,
        pltpu.SemaphoreType.DMA,
        pltpu.SemaphoreType.DMA,
    ],
  )


def kernel(x, edge_index, batch, W_emb, b_emb, W_att, a_att, W_out, b_out):
    xp = jnp.pad(x, ((0, NP - N), (0, 0)))
    src = jnp.concatenate([edge_index[0],
                           jnp.full((EPAD - E,), N, _i32)]).reshape(EROWS, CH)
    dst = jnp.concatenate([edge_index[1],
                           jnp.zeros((EPAD - E,), _i32)]).reshape(EROWS, CH)
    batchp = jnp.pad(batch, (0, NP - N)).reshape(NP, 1)

    va = jnp.einsum('kij,kj->ki', W_att, a_att[:, :HID])
    vd = jnp.einsum('kij,kj->ki', W_att, a_att[:, HID:])
    Wbig = jnp.stack([jnp.concatenate(
        [W_att[2 * c], W_att[2 * c + 1]], axis=1)
        for c in range(2)])
    Vbig = jnp.stack([jnp.stack(
        [va[2 * c], va[2 * c + 1], vd[2 * c], vd[2 * c + 1]], axis=1)
        for c in range(2)])

    xc = _embed(xp, W_emb, b_emb)
    yc = xc
    for step in range(NSTEPS):
        hc, ss = _phase_a(xc, Wbig, Vbig)
        wv, rs = _make_sc_w()(ss.reshape(2, NP * 4), src, dst)
        acc = _make_sc_agg()(hc.reshape(2 * NP, RW), wv, src, dst)
        den = rs.reshape(2, NP, 2)
        xc, yc = _phase_c(acc, den, xc, yc)
    pooled = _phase_d(yc, W_out, b_out, batchp)
    return pooled[0]
